# baseline (speedup 1.0000x reference)
import jax
import jax.numpy as jnp
from jax import lax
from jax.experimental import pallas as pl
from jax.experimental.pallas import tpu as pltpu
from jax.experimental.pallas import tpu_sc as plsc

_B, _T, _D = 4, 8192, 1024
_NS = 8
_ME = 32
_THR = 2.0
_MINEV = 4
_TV = 512
_PARTS = 8
_CHUNK = _T // _PARTS
_NV = _CHUNK // 16


def _sortd(v):
    r = plsc.sort_key_val(v, v, descending=True)
    return r[0] if isinstance(r, (tuple, list)) else r


def _rev(v):
    return lax.rev(v, (0,))


def _merge32_16(t0, t1, x):
    xs = _sortd(x)
    u = _sortd(jnp.maximum(t1, _rev(xs)))
    hi = jnp.maximum(t0, _rev(u))
    lo = jnp.minimum(t0, _rev(u))
    return _sortd(hi), _sortd(lo)


def _splat_i32(x):
    return jnp.broadcast_to(x, (16,)).astype(jnp.int32)


def _sc_select_gather(z_ref, h_ref, tt_ref,
                      times_out, mask_out, kvec_out, rows_out, ttrows_out,
                      hx_out,
                      hx_out,
                      zv, sv, st1, st2, kst, a1, a2, fin, mbuf, idxv, hrows,
                      ttrv, sh1, sh2, semh, semt):
    c = lax.axis_index("c")
    s = lax.axis_index("s")
    batch = 2 * c + s // _PARTS
    part = s % _PARTS
    grp = (s // _PARTS) * _PARTS
    iota = lax.broadcasted_iota(jnp.int32, (16,), 0)

    for w in range(_NS):
        pltpu.sync_copy(z_ref.at[batch, w, part],
                        zv.at[pl.ds(w * _CHUNK, _CHUNK)])

    neg = jnp.full((16,), -1.0, jnp.float32)

    def abody(o, carry):
        t0, t1, cnt = carry
        base = o * 16 + iota
        x = jnp.abs(plsc.load_gather(zv, [base]))
        for w in range(1, _NS):
            x = jnp.maximum(x, jnp.abs(plsc.load_gather(zv,
                                                        [w * _CHUNK + base])))
        plsc.store_scatter(sv, [base], x)
        cnt = cnt + jnp.sum((x > _THR).astype(jnp.int32))
        t1min = jnp.min(t1)
        t0, t1 = lax.cond(jnp.max(x) > t1min,
                          lambda a: _merge32_16(a[0], a[1], a[2]),
                          lambda a: (a[0], a[1]),
                          (t0, t1, x))
        return t0, t1, cnt

    t0, t1, cnt = lax.fori_loop(0, _NV, abody, (neg, neg, jnp.int32(0)))
    st1[pl.ds(0, 16)] = t0
    st1[pl.ds(16, 16)] = t1
    st1[pl.ds(32, 16)] = cnt.astype(jnp.float32) + jnp.zeros((16,), jnp.float32)
    st1[pl.ds(48, 16)] = jnp.zeros((16,), jnp.float32)
    pltpu.sync_copy(st1, sh1.at[s])
    plsc.subcore_barrier()

    pltpu.sync_copy(sh1, a1)

    def _group_merge(rlo):
        m0 = a1[rlo, pl.ds(0, 16)]
        m1 = a1[rlo, pl.ds(16, 16)]
        na = jnp.max(a1[rlo, pl.ds(32, 16)])
        for r in range(rlo + 1, rlo + _PARTS):
            m0, m1 = _merge32_16(m0, m1, a1[r, pl.ds(0, 16)])
            m0, m1 = _merge32_16(m0, m1, a1[r, pl.ds(16, 16)])
            na = na + jnp.max(a1[r, pl.ds(32, 16)])
        return m0, m1, na

    mA0, mA1, naA = _group_merge(0)
    mB0, mB1, naB = _group_merge(_PARTS)
    in_b = grp > 0
    m0 = jnp.where(jnp.broadcast_to(in_b, (16,)), mB0, mA0)
    m1 = jnp.where(jnp.broadcast_to(in_b, (16,)), mB1, mA1)
    n_above = jnp.where(in_b, naB, naA).astype(jnp.int32)
    k = jnp.where(n_above < _MINEV, _ME, jnp.minimum(n_above, _ME))
    s0 = jnp.sum(jnp.where(iota == (k - 1), m0, 0.0))
    s1 = jnp.sum(jnp.where(iota == (k - 17), m1, 0.0))
    vk = jnp.where(k <= 16, s0, s1)

    def cbody(o, carry):
        cg, ce = carry
        x = plsc.load_gather(sv, [o * 16 + iota])
        cg = cg + jnp.sum((x > vk).astype(jnp.int32))
        ce = ce + jnp.sum((x == vk).astype(jnp.int32))
        return cg, ce

    cg, ce = lax.fori_loop(0, _NV, cbody, (jnp.int32(0), jnp.int32(0)))
    st2[pl.ds(0, 16)] = _splat_i32(cg)
    st2[pl.ds(16, 16)] = _splat_i32(ce)
    st2[pl.ds(32, 16)] = jnp.zeros((16,), jnp.int32)
    st2[pl.ds(48, 16)] = jnp.zeros((16,), jnp.int32)
    pltpu.sync_copy(st2, sh2.at[s])
    plsc.subcore_barrier()

    pltpu.sync_copy(sh2, a2)
    zero16 = jnp.zeros((16,), jnp.int32)
    cgt_v = zero16
    ceq_v = zero16
    for r in range(16):
        lane = r - grp
        cgt_v = cgt_v + jnp.where(iota == lane,
                                  jnp.max(a2[r, pl.ds(0, 16)]), 0)
        ceq_v = ceq_v + jnp.where(iota == lane,
                                  jnp.max(a2[r, pl.ds(16, 16)]), 0)
    c1 = jnp.sum(jnp.where(iota < _PARTS, cgt_v, 0))
    needed = k - c1
    cum_eq_excl = plsc.cumsum(ceq_v) - ceq_v
    take_v = jnp.clip(jnp.broadcast_to(needed, (16,)) - cum_eq_excl, 0, ceq_v)
    take_p = jnp.sum(jnp.where(iota == part, take_v, 0))

    fin[pl.ds(0, 16)] = zero16
    fin[pl.ds(16, 16)] = zero16
    tbase = part * _CHUNK

    def dbody(o, carry):
        eq_sofar, sel_sofar = carry
        x = plsc.load_gather(sv, [o * 16 + iota])
        m_gt = x > vk
        m_eq = x == vk
        eqc = plsc.cumsum(m_eq.astype(jnp.int32))
        sel = m_gt | (m_eq & ((eq_sofar + eqc) <= take_p))
        sc = plsc.cumsum(sel.astype(jnp.int32))
        dest = sel_sofar + sc - 1
        tvec = tbase + o * 16 + iota
        plsc.store_scatter(fin, [dest], tvec, mask=sel & (dest < _ME))
        return (eq_sofar + jnp.sum(m_eq.astype(jnp.int32)),
                sel_sofar + jnp.sum(sel.astype(jnp.int32)))

    _, nsel = lax.fori_loop(0, _NV, dbody, (jnp.int32(0), jnp.int32(0)))
    st2[pl.ds(0, 16)] = plsc.load_gather(fin, [iota])
    st2[pl.ds(16, 16)] = plsc.load_gather(fin, [16 + iota])
    st2[pl.ds(32, 16)] = _splat_i32(nsel)
    st2[pl.ds(48, 16)] = zero16
    pltpu.sync_copy(st2, hx_out.at[c * 16 + s])
    plsc.subcore_barrier()

    @pl.when(part == 0)
    def _():
        pltpu.sync_copy(hx_out.at[pl.ds(c * 16, 16)], a2)
        fin[pl.ds(0, 16)] = zero16
        fin[pl.ds(16, 16)] = zero16
        off = jnp.int32(0)
        for r in range(16):
            gate = (r >= grp) & (r < grp + _PARTS)
            gv = jnp.broadcast_to(gate, (16,))
            np_ = jnp.max(a2[r, pl.ds(32, 16)])
            v0 = a2[r, pl.ds(0, 16)]
            v1 = a2[r, pl.ds(16, 16)]
            plsc.store_scatter(fin, [off + iota], v0,
                               mask=gv & (iota < np_) & ((off + iota) < _ME))
            plsc.store_scatter(fin, [off + 16 + iota], v1,
                               mask=gv & ((iota + 16) < np_)
                               & ((off + 16 + iota) < _ME))
            off = off + jnp.where(gate, np_, 0)
        pltpu.sync_copy(fin, times_out.at[batch])
        mbuf[pl.ds(0, 16)] = (iota < k).astype(jnp.int32)
        mbuf[pl.ds(16, 16)] = ((iota + 16) < k).astype(jnp.int32)
        pltpu.sync_copy(mbuf, mask_out.at[batch])
        kst[...] = _splat_i32(k)
        pltpu.sync_copy(kst, kvec_out.at[batch])
        f0 = plsc.load_gather(fin, [iota])
        f1 = plsc.load_gather(fin, [16 + iota])
        plsc.store_scatter(idxv, [iota], batch * _T + f0)
        plsc.store_scatter(idxv, [16 + iota], batch * _T + f1)
        pltpu.async_copy(h_ref.at[idxv], hrows, semh).wait()
        pltpu.sync_copy(hrows, rows_out.at[pl.ds(batch * _ME, _ME)])
        plsc.store_scatter(idxv, [iota], jnp.minimum(f0, _TV - 1))
        plsc.store_scatter(idxv, [16 + iota], jnp.minimum(f1, _TV - 1))
        pltpu.async_copy(tt_ref.at[idxv], ttrv, semt).wait()
        pltpu.sync_copy(ttrv, ttrows_out.at[pl.ds(batch * _ME, _ME)])


def _sc_call(z_r, h2d, tt):
    mesh = plsc.VectorSubcoreMesh(core_axis_name="c", subcore_axis_name="s")
    f = pl.kernel(
        _sc_select_gather,
        out_type=[
            jax.ShapeDtypeStruct((_B, _ME), jnp.int32),
            jax.ShapeDtypeStruct((_B, _ME), jnp.int32),
            jax.ShapeDtypeStruct((_B, 16), jnp.int32),
            jax.ShapeDtypeStruct((_B * _ME, _D), jnp.float32),
            jax.ShapeDtypeStruct((_B * _ME, _D), jnp.float32),
            jax.ShapeDtypeStruct((32, 64), jnp.int32),
        ],
        mesh=mesh,
        compiler_params=pltpu.CompilerParams(needs_layout_passes=False),
        scratch_types=[
            pltpu.VMEM((_NS * _CHUNK,), jnp.float32),
            pltpu.VMEM((_CHUNK,), jnp.float32),
            pltpu.VMEM((64,), jnp.float32),
            pltpu.VMEM((64,), jnp.int32),
            pltpu.VMEM((16,), jnp.int32),
            pltpu.VMEM((16, 64), jnp.float32),
            pltpu.VMEM((16, 64), jnp.int32),
            pltpu.VMEM((_ME,), jnp.int32),
            pltpu.VMEM((_ME,), jnp.int32),
            pltpu.VMEM((_ME,), jnp.int32),
            pltpu.VMEM((_ME, _D), jnp.float32),
            pltpu.VMEM((_ME, _D), jnp.float32),
            pltpu.VMEM_SHARED((16, 64), jnp.float32),
            pltpu.VMEM_SHARED((16, 64), jnp.int32),
            pltpu.SemaphoreType.DMA,
            pltpu.SemaphoreType.DMA,
        ],
    )
    return f(z_r, h2d, tt)


def _tc_project(rows_ref, ttrows_ref, W_ref, b_ref, k_ref, out_ref):
    parts = []
    for b in range(_B):
        kb = k_ref[b, 0]
        rb = rows_ref[pl.ds(b * _ME, _ME), :]
        gated = jnp.where(
            jax.lax.broadcasted_iota(jnp.int32, (_ME, _D), 0) < kb, rb, 0.0)
        parts.append(gated)
    gated_all = jnp.concatenate(parts, axis=0)
    entries = (jax.lax.dot_general(gated_all, W_ref[...],
                                   (((1,), (1,)), ((), ())),
                                   preferred_element_type=jnp.float32)
               + b_ref[...] + ttrows_ref[...])
    out_ref[...] = entries.reshape(_B, _ME, _D)


def kernel(h_seq, z_per_step, W, b, time_table):
    z_r = z_per_step.transpose(0, 2, 1).reshape(_B, _NS, _PARTS, _CHUNK)
    h2d = h_seq.reshape(_B * _T, _D)
    times, mask_i, kvec, rows, ttrows, _hx = _sc_call(z_r, h2d, time_table)
    entries = pl.pallas_call(
        _tc_project,
        grid=(1,),
        in_specs=[
            pl.BlockSpec((_B * _ME, _D), lambda i: (0, 0)),
            pl.BlockSpec((_B * _ME, _D), lambda i: (0, 0)),
            pl.BlockSpec((_D, _D), lambda i: (0, 0)),
            pl.BlockSpec((1, _D), lambda i: (0, 0)),
            pl.BlockSpec(memory_space=pltpu.SMEM),
        ],
        out_specs=pl.BlockSpec((_B, _ME, _D), lambda i: (0, 0, 0)),
        out_shape=jax.ShapeDtypeStruct((_B, _ME, _D), jnp.float32),
    )(rows, ttrows, W, b.reshape(1, _D), kvec)
    return entries, mask_i.astype(bool), times


# speedup vs baseline: 1.0663x; 1.0663x over previous
import jax
import jax.numpy as jnp
from jax import lax
from jax.experimental import pallas as pl
from jax.experimental.pallas import tpu as pltpu
from jax.experimental.pallas import tpu_sc as plsc

_B, _T, _D = 4, 8192, 1024
_NS = 8
_ME = 32
_THR = 2.0
_MINEV = 4
_TV = 512
_TU = _T // 8


def _select_kernel(z_ref, mask_ref, times_ref, kvec_ref):
    s = jnp.max(jnp.abs(z_ref[...]), axis=1)
    it = (jax.lax.broadcasted_iota(jnp.int32, (_B, 8, _TU), 1) * _TU
          + jax.lax.broadcasted_iota(jnp.int32, (_B, 8, _TU), 2))

    n_above = jnp.sum(
        jnp.sum((s > _THR).astype(jnp.int32), axis=2, keepdims=True),
        axis=1, keepdims=True)
    k = jnp.where(n_above < _MINEV, _ME, jnp.minimum(n_above, _ME))

    pos3 = jax.lax.broadcasted_iota(jnp.int32, (_B, 1, _ME), 2)

    def body(j, carry):
        x, tacc = carry
        m = jnp.max(jnp.max(x, axis=2, keepdims=True), axis=1, keepdims=True)
        cand = jnp.where(x == jnp.broadcast_to(m, (_B, 8, _TU)), it, _T)
        iv = jnp.min(jnp.min(cand, axis=2, keepdims=True), axis=1,
                     keepdims=True)
        tacc = jnp.where(pos3 == j, jnp.broadcast_to(iv, (_B, 1, _ME)), tacc)
        x = jnp.where(it == jnp.broadcast_to(iv, (_B, 8, _TU)), -1.0, x)
        return x, tacc

    _, times_v = jax.lax.fori_loop(
        0, _ME, body, (s, jnp.zeros((_B, 1, _ME), jnp.int32)))

    eye = (jax.lax.broadcasted_iota(jnp.int32, (_ME, _ME), 0)
           == jax.lax.broadcasted_iota(jnp.int32, (_ME, _ME), 1)
           ).astype(jnp.float32)

    def tmul(a):
        return jax.lax.dot_general(a, eye, (((0,), (0,)), ((), ())),
                                   preferred_element_type=jnp.float32,
                                   precision=jax.lax.Precision.HIGHEST)

    pos_row = jax.lax.broadcasted_iota(jnp.int32, (1, _ME), 1)
    tsi_parts, valid_parts = [], []
    for b in range(_B):
        tv = times_v[b]
        kb = k[b]
        valid = pos_row < jnp.broadcast_to(kb, (1, _ME))
        tprime = jnp.where(valid, tv, _T).astype(jnp.float32)
        tp_cols = jnp.broadcast_to(tprime, (_ME, _ME))
        tp_rows = tmul(tp_cols)
        rank = jnp.sum((tp_rows < tp_cols).astype(jnp.float32), axis=0,
                       keepdims=True)
        rank_rows = tmul(jnp.broadcast_to(rank, (_ME, _ME)))
        q = ((rank_rows.astype(jnp.int32)
              == jax.lax.broadcasted_iota(jnp.int32, (_ME, _ME), 1))
             & (jax.lax.broadcasted_iota(jnp.int32, (_ME, _ME), 0)
                < jnp.broadcast_to(kb, (_ME, _ME)))
             ).astype(jnp.float32)
        tsorted = jax.lax.dot_general(tv.astype(jnp.float32), q,
                                      (((1,), (0,)), ((), ())),
                                      preferred_element_type=jnp.float32,
                                      precision=jax.lax.Precision.HIGHEST)
        tsi_parts.append(tsorted.astype(jnp.int32))
        valid_parts.append(valid.astype(jnp.int32))

    tsi = jnp.concatenate(tsi_parts, axis=0)
    valid_all = jnp.concatenate(valid_parts, axis=0)
    kb_all = jnp.concatenate([jnp.broadcast_to(k[b], (1, _ME))
                              for b in range(_B)], axis=0)

    mask_ref[...] = valid_all[:, None, :]
    times_ref[...] = tsi[:, None, :]
    kvec_ref[...] = kb_all[:, None, :]


def _sc_gather(times_ref, h_ref, tt_ref, rows_out, ttrows_out,
               fin, idxv, hrows, ttrv, semh, semt):
    c = lax.axis_index("c")
    s = lax.axis_index("s")
    batch = 2 * c + s // 8
    part = s % 8
    iota = lax.broadcasted_iota(jnp.int32, (16,), 0)

    @pl.when(part == 0)
    def _():
        pltpu.sync_copy(times_ref.at[batch], fin)
        f0 = plsc.load_gather(fin, [iota])
        f1 = plsc.load_gather(fin, [16 + iota])
        plsc.store_scatter(idxv, [iota], batch * _T + f0)
        plsc.store_scatter(idxv, [16 + iota], batch * _T + f1)
        pltpu.async_copy(h_ref.at[idxv], hrows, semh).wait()
        pltpu.sync_copy(hrows, rows_out.at[pl.ds(batch * _ME, _ME)])
        plsc.store_scatter(idxv, [iota], jnp.minimum(f0, _TV - 1))
        plsc.store_scatter(idxv, [16 + iota], jnp.minimum(f1, _TV - 1))
        pltpu.async_copy(tt_ref.at[idxv], ttrv, semt).wait()
        pltpu.sync_copy(ttrv, ttrows_out.at[pl.ds(batch * _ME, _ME)])


def _sc_gather_call(times2, h2d, tt):
    mesh = plsc.VectorSubcoreMesh(core_axis_name="c", subcore_axis_name="s")
    f = pl.kernel(
        _sc_gather,
        out_type=[
            jax.ShapeDtypeStruct((_B * _ME, _D), jnp.float32),
            jax.ShapeDtypeStruct((_B * _ME, _D), jnp.float32),
        ],
        mesh=mesh,
        compiler_params=pltpu.CompilerParams(needs_layout_passes=False),
        scratch_types=[
            pltpu.VMEM((_ME,), jnp.int32),
            pltpu.VMEM((_ME,), jnp.int32),
            pltpu.VMEM((_ME, _D), jnp.float32),
            pltpu.VMEM((_ME, _D), jnp.float32),
            pltpu.SemaphoreType.DMA,
            pltpu.SemaphoreType.DMA,
        ],
    )
    return f(times2, h2d, tt)


def _tc_project(rows_ref, ttrows_ref, W_ref, b_ref, k_ref, out_ref):
    parts = []
    for b in range(_B):
        kb = k_ref[b, 0]
        rb = rows_ref[pl.ds(b * _ME, _ME), :]
        gated = jnp.where(
            jax.lax.broadcasted_iota(jnp.int32, (_ME, _D), 0) < kb, rb, 0.0)
        parts.append(gated)
    gated_all = jnp.concatenate(parts, axis=0)
    entries = (jax.lax.dot_general(gated_all, W_ref[...],
                                   (((1,), (1,)), ((), ())),
                                   preferred_element_type=jnp.float32)
               + b_ref[...] + ttrows_ref[...])
    out_ref[...] = entries.reshape(_B, _ME, _D)


def kernel(h_seq, z_per_step, W, b, time_table):
    z_r = z_per_step.transpose(0, 2, 1).reshape(_B, _NS, 8, _TU)
    mask_i, times, kvec = pl.pallas_call(
        _select_kernel,
        grid=(1,),
        in_specs=[pl.BlockSpec((_B, _NS, 8, _TU), lambda i: (0, 0, 0, 0))],
        out_specs=[
            pl.BlockSpec((_B, 1, _ME), lambda i: (0, 0, 0)),
            pl.BlockSpec((_B, 1, _ME), lambda i: (0, 0, 0)),
            pl.BlockSpec((_B, 1, _ME), lambda i: (0, 0, 0)),
        ],
        out_shape=[
            jax.ShapeDtypeStruct((_B, 1, _ME), jnp.int32),
            jax.ShapeDtypeStruct((_B, 1, _ME), jnp.int32),
            jax.ShapeDtypeStruct((_B, 1, _ME), jnp.int32),
        ],
    )(z_r)
    times2 = times.reshape(_B, _ME)
    h2d = h_seq.reshape(_B * _T, _D)
    rows, ttrows = _sc_gather_call(times2, h2d, time_table)
    entries = pl.pallas_call(
        _tc_project,
        grid=(1,),
        in_specs=[
            pl.BlockSpec((_B * _ME, _D), lambda i: (0, 0)),
            pl.BlockSpec((_B * _ME, _D), lambda i: (0, 0)),
            pl.BlockSpec((_D, _D), lambda i: (0, 0)),
            pl.BlockSpec((1, _D), lambda i: (0, 0)),
            pl.BlockSpec(memory_space=pltpu.SMEM),
        ],
        out_specs=pl.BlockSpec((_B, _ME, _D), lambda i: (0, 0, 0)),
        out_shape=jax.ShapeDtypeStruct((_B, _ME, _D), jnp.float32),
    )(rows, ttrows, W, b.reshape(1, _D), kvec.reshape(_B, _ME))
    return entries, mask_i.reshape(_B, _ME).astype(bool), times2
